# baseline (speedup 1.0000x reference)
import jax
import jax.numpy as jnp
from jax.experimental import pallas as pl

_TILE = 2048


def _fused_mlp_kernel(x_ref, w1_ref, b1_ref, w2_ref, b2_ref, q_ref, o_ref):
    h = jnp.dot(x_ref[...], w1_ref[...], preferred_element_type=jnp.float32)
    h = jnp.maximum(h + b1_ref[...], 0.0)
    h = jnp.dot(h, w2_ref[...], preferred_element_type=jnp.float32)
    h = jnp.maximum(h + b2_ref[...], 0.0)
    o_ref[...] = jax.lax.dot_general(
        q_ref[...], h,
        dimension_numbers=(((1,), (1,)), ((), ())),
        preferred_element_type=jnp.float32,
    )


def kernel(coord, feat, batch, w1, b1, w2, b2, queries):
    x = jnp.concatenate([feat, coord], axis=-1)
    n, d_in = x.shape
    embed = w1.shape[1]
    k = queries.shape[0]
    tile = _TILE if n % _TILE == 0 else n
    grid = n // tile

    masks = pl.pallas_call(
        _fused_mlp_kernel,
        grid=(grid,),
        in_specs=[
            pl.BlockSpec((tile, d_in), lambda i: (i, 0)),
            pl.BlockSpec((d_in, embed), lambda i: (0, 0)),
            pl.BlockSpec((1, embed), lambda i: (0, 0)),
            pl.BlockSpec((embed, embed), lambda i: (0, 0)),
            pl.BlockSpec((1, embed), lambda i: (0, 0)),
            pl.BlockSpec((k, embed), lambda i: (0, 0)),
        ],
        out_specs=pl.BlockSpec((k, tile), lambda i: (0, i)),
        out_shape=jax.ShapeDtypeStruct((k, n), jnp.float32),
    )(x, w1, b1[None, :], w2, b2[None, :], queries)
    return masks, batch
